# baseline (speedup 1.0000x reference)
import functools

import jax
import jax.numpy as jnp
from jax import lax
from jax.experimental import pallas as pl
from jax.experimental.pallas import tpu as pltpu
from jax.experimental.pallas import tpu_sc as plsc

N = 10000
E = 320000
D = 128
NP = 10240
NC = 2
NS = 16
L = 16
NW = NC * NS
EPW = E // NW
BE = 80
NCHUNK = EPW // BE
NPT = NP // NW
BKT = 512
BKE = BKT - 1
BEC = 2000
NCH_P = EPW // BEC
BV = 128
RB = 512
GRID = NP // RB
RSQRT_C = 1.0 / (128.0 ** 0.5)

_mesh = plsc.VectorSubcoreMesh(core_axis_name="c", subcore_axis_name="s")


def _permute(v, idx):
  return v.at[idx].get(mode="promise_in_bounds")


def _splat_sum(v, lane):
  for sh in (8, 4, 2, 1):
    v = v + _permute(v, lane ^ sh)
  return v


def _splat_max(v, lane):
  for sh in (8, 4, 2, 1):
    v = jnp.maximum(v, _permute(v, lane ^ sh))
  return v


def _build_sc_alpha():
  @functools.partial(
      pl.kernel,
      out_type=[jax.ShapeDtypeStruct((E,), jnp.float32),
                jax.ShapeDtypeStruct((NW, L), jnp.float32)],
      mesh=_mesh,
      scratch_types=[
          pltpu.VMEM((2, BE), jnp.int32),
          pltpu.VMEM((2, BE), jnp.int32),
          pltpu.VMEM((2, BE, D), jnp.float32),
          pltpu.VMEM((2, BE, D), jnp.float32),
          pltpu.VMEM((BE,), jnp.float32),
          pltpu.VMEM((L,), jnp.float32),
          pltpu.SemaphoreType.DMA((2,)),
          pltpu.SemaphoreType.DMA((2,)),
      ],
  )
  def sc_alpha(src_hbm, dst_hbm, q_hbm, k_hbm, alpha_hbm, mx_hbm,
               srcv, dstv, qv, kv, av, mxv, semq, semk):
    c = lax.axis_index("c")
    s = lax.axis_index("s")
    wid = s * NC + c
    base = wid * EPW

    def fetch(i, b):
      off = base + i * BE
      pltpu.sync_copy(src_hbm.at[pl.ds(off, BE)], srcv.at[b])
      pltpu.sync_copy(dst_hbm.at[pl.ds(off, BE)], dstv.at[b])
      pltpu.async_copy(q_hbm.at[dstv.at[b]], qv.at[b], semq.at[b])
      pltpu.async_copy(k_hbm.at[srcv.at[b]], kv.at[b], semk.at[b])

    fetch(0, 0)
    mxv[...] = jnp.full((L,), -jnp.inf, jnp.float32)
    lane = lax.iota(jnp.int32, L)

    def compute(bs):
      def group(g, _):
        alpha16 = jnp.zeros((L,), jnp.float32)
        for jj in range(L):
          j = g * L + jj
          qrow = qv.at[bs, j]
          krow = kv.at[bs, j]
          acc = qrow[pl.ds(0, L)] * krow[pl.ds(0, L)]
          for t in range(1, D // L):
            acc = acc + qrow[pl.ds(t * L, L)] * krow[pl.ds(t * L, L)]
          a = _splat_sum(acc, lane) * RSQRT_C
          alpha16 = jnp.where(lane == jj, a, alpha16)
        av[pl.ds(g * L, L)] = alpha16
        mxv[...] = jnp.maximum(mxv[...], alpha16)
        return 0

      lax.fori_loop(0, BE // L, group, 0)

    def chunk(i, _):
      b = i % 2

      @pl.when(i + 1 < NCHUNK)
      def _():
        fetch(i + 1, (i + 1) % 2)

      @pl.when(b == 0)
      def _():
        pltpu.make_async_copy(q_hbm.at[dstv.at[0]], qv.at[0],
                              semq.at[0]).wait()
        pltpu.make_async_copy(k_hbm.at[srcv.at[0]], kv.at[0],
                              semk.at[0]).wait()
        compute(0)

      @pl.when(b == 1)
      def _():
        pltpu.make_async_copy(q_hbm.at[dstv.at[1]], qv.at[1],
                              semq.at[1]).wait()
        pltpu.make_async_copy(k_hbm.at[srcv.at[1]], kv.at[1],
                              semk.at[1]).wait()
        compute(1)

      pltpu.sync_copy(av, alpha_hbm.at[pl.ds(base + i * BE, BE)])
      return 0

    lax.fori_loop(0, NCHUNK, chunk, 0)
    pltpu.sync_copy(mxv, mx_hbm.at[wid])

  return sc_alpha


def _build_sc_partition():
  @functools.partial(
      pl.kernel,
      out_type=[jax.ShapeDtypeStruct((NW, NW, BKT), jnp.int32),
                jax.ShapeDtypeStruct((NW, NW, BKT), jnp.int32),
                jax.ShapeDtypeStruct((NW, NW, BKT), jnp.int32),
                jax.ShapeDtypeStruct((NW * NW * L,), jnp.int32)],
      mesh=_mesh,
      scratch_types=[
          pltpu.VMEM((BEC,), jnp.int32),
          pltpu.VMEM((BEC,), jnp.int32),
          pltpu.VMEM((NW, BKT), jnp.int32),
          pltpu.VMEM((NW, BKT), jnp.int32),
          pltpu.VMEM((NW, BKT), jnp.int32),
          pltpu.VMEM((NW * L,), jnp.int32),
          pltpu.SMEM((NW,), jnp.int32),
      ],
  )
  def sc_partition(src_hbm, dst_hbm, psrc_hbm, pdl_hbm, peid_hbm, cnt_hbm,
                   srcv, dstv, bsrc, bdl, beid, cntv, cnts):
    c = lax.axis_index("c")
    s = lax.axis_index("s")
    wid = s * NC + c
    base = wid * EPW

    def zcnt(o, _):
      cnts[o] = 0
      return 0

    lax.fori_loop(0, NW, zcnt, 0)

    def zfill(i, _):
      r = i // (BKT // L)
      t = i % (BKT // L)
      z = jnp.zeros((L,), jnp.int32)
      bsrc.at[r][pl.ds(t * L, L)] = z
      bdl.at[r][pl.ds(t * L, L)] = z
      beid.at[r][pl.ds(t * L, L)] = z
      return 0

    lax.fori_loop(0, NW * (BKT // L), zfill, 0)

    def chunk(i, _):
      off = base + i * BEC
      pltpu.sync_copy(src_hbm.at[pl.ds(off, BEC)], srcv)
      pltpu.sync_copy(dst_hbm.at[pl.ds(off, BEC)], dstv)

      def group(g, _):
        d16 = dstv[pl.ds(g * L, L)]
        s16 = srcv[pl.ds(g * L, L)]
        lane = lax.iota(jnp.int32, L)
        for jj in range(L):
          dj = d16[jj]
          o = dj // NPT
          dl = dj - o * NPT
          cnt = jnp.minimum(cnts[o], BKE)
          eid = off + g * L + jj
          gb = pl.multiple_of((cnt // L) * L, L)
          sel = lane == (cnt - gb)
          srow = bsrc.at[o]
          srow[pl.ds(gb, L)] = jnp.where(
              sel, jnp.full((L,), s16[jj], jnp.int32), srow[pl.ds(gb, L)])
          drow = bdl.at[o]
          drow[pl.ds(gb, L)] = jnp.where(
              sel, jnp.full((L,), dl, jnp.int32), drow[pl.ds(gb, L)])
          erow = beid.at[o]
          erow[pl.ds(gb, L)] = jnp.where(
              sel, jnp.full((L,), eid, jnp.int32), erow[pl.ds(gb, L)])
          cnts[o] = cnt + 1
        return 0

      lax.fori_loop(0, BEC // L, group, 0)
      return 0

    lax.fori_loop(0, NCH_P, chunk, 0)

    def emit(o, _):
      cntv[pl.ds(o * L, L)] = jnp.full(
          (L,), jnp.minimum(cnts[o], BKE), jnp.int32)
      return 0

    lax.fori_loop(0, NW, emit, 0)
    pltpu.sync_copy(bsrc, psrc_hbm.at[wid])
    pltpu.sync_copy(bdl, pdl_hbm.at[wid])
    pltpu.sync_copy(beid, peid_hbm.at[wid])
    pltpu.sync_copy(cntv, cnt_hbm.at[pl.ds(wid * NW * L, NW * L)])

  return sc_partition


def _build_sc_accum():
  @functools.partial(
      pl.kernel,
      out_type=[jax.ShapeDtypeStruct((NP, D), jnp.float32),
                jax.ShapeDtypeStruct((NP * L,), jnp.float32)],
      mesh=_mesh,
      scratch_types=[
          pltpu.VMEM((2 * BKT,), jnp.int32),
          pltpu.VMEM((2 * BKT,), jnp.int32),
          pltpu.VMEM((2 * BKT,), jnp.int32),
          pltpu.VMEM((NW * NW * L,), jnp.int32),
          pltpu.VMEM((NW, L), jnp.float32),
          pltpu.VMEM((2, BV, D), jnp.float32),
          pltpu.VMEM((2 * BV,), jnp.float32),
          pltpu.VMEM((NPT, D), jnp.float32),
          pltpu.VMEM((NPT * L,), jnp.float32),
          pltpu.SemaphoreType.DMA((2,)),
          pltpu.SemaphoreType.DMA((2,)),
          pltpu.SemaphoreType.DMA((2,)),
      ],
  )
  def sc_accum(psrc_hbm, pdl_hbm, peid_hbm, cnt_hbm, mx_hbm, alpha_hbm, v_hbm,
               num_hbm, den_hbm,
               csrc, cdl, ceid, cntv, mxall, vv, avb, numloc, denloc,
               semr, sema, semv):
    c = lax.axis_index("c")
    s = lax.axis_index("s")
    wid = s * NC + c
    lane = lax.iota(jnp.int32, L)

    pltpu.sync_copy(cnt_hbm, cntv)
    pltpu.sync_copy(mx_hbm, mxall)

    def mred(i, mv):
      return jnp.maximum(mv, mxall[i])

    mv = lax.fori_loop(0, NW, mred, jnp.full((L,), -jnp.inf, jnp.float32))
    m = _splat_max(mv, lane)

    def znum(i, _):
      r = i // (D // L)
      t = i % (D // L)
      numloc.at[r][pl.ds(t * L, L)] = jnp.zeros((L,), jnp.float32)
      return 0

    lax.fori_loop(0, NPT * (D // L), znum, 0)

    def zden(i, _):
      denloc[pl.ds(i * L, L)] = jnp.zeros((L,), jnp.float32)
      return 0

    lax.fori_loop(0, NPT, zden, 0)

    def fetch_rows(sc, b):
      bo = pl.multiple_of(b * BKT, BKT)
      pltpu.async_copy(psrc_hbm.at[sc, wid], csrc.at[pl.ds(bo, BKT)],
                       semr.at[b])
      pltpu.async_copy(pdl_hbm.at[sc, wid], cdl.at[pl.ds(bo, BKT)],
                       semr.at[b])
      pltpu.async_copy(peid_hbm.at[sc, wid], ceid.at[pl.ds(bo, BKT)],
                       semr.at[b])

    def wait_rows(sc, b):
      bo = pl.multiple_of(b * BKT, BKT)
      pltpu.make_async_copy(psrc_hbm.at[sc, wid], csrc.at[pl.ds(bo, BKT)],
                            semr.at[b]).wait()
      pltpu.make_async_copy(pdl_hbm.at[sc, wid], cdl.at[pl.ds(bo, BKT)],
                            semr.at[b]).wait()
      pltpu.make_async_copy(peid_hbm.at[sc, wid], ceid.at[pl.ds(bo, BKT)],
                            semr.at[b]).wait()

    fetch_rows(0, 0)

    def scanner(sc, _):
      br = sc % 2

      @pl.when(sc + 1 < NW)
      def _():
        fetch_rows(sc + 1, (sc + 1) % 2)

      wait_rows(sc, br)
      co = pl.multiple_of((sc * NW + wid) * L, L)
      cnt16 = cntv[pl.ds(co, L)]
      cnt = cnt16[0]
      nch = (cnt + (BV - 1)) // BV

      def fetch_chunk(i, b):
        ro = pl.multiple_of(br * BKT + i * BV, BV)
        ao = pl.multiple_of(b * BV, BV)
        pltpu.async_copy(alpha_hbm.at[ceid.at[pl.ds(ro, BV)]],
                         avb.at[pl.ds(ao, BV)], sema.at[b])
        pltpu.async_copy(v_hbm.at[csrc.at[pl.ds(ro, BV)]],
                         vv.at[b], semv.at[b])

      @pl.when(nch > 0)
      def _():
        fetch_chunk(0, 0)

      def do_chunk(i, bs):
        off = i * BV
        ro = pl.multiple_of(br * BKT + off, BV)
        pltpu.make_async_copy(alpha_hbm.at[ceid.at[pl.ds(ro, BV)]],
                              avb.at[pl.ds(bs * BV, BV)],
                              sema.at[bs]).wait()
        pltpu.make_async_copy(v_hbm.at[csrc.at[pl.ds(ro, BV)]],
                              vv.at[bs], semv.at[bs]).wait()

        def group(g, _):
          a16 = avb[pl.ds(bs * BV + g * L, L)]
          idx16 = jnp.full((L,), off + g * L, jnp.int32) + lane
          live = idx16 < cnt16
          e16 = jnp.where(live, jnp.exp(a16 - m), 0.0)
          dl16 = cdl[pl.ds(br * BKT + off + g * L, L)]
          for jj in range(L):
            j = g * L + jj
            ej = _permute(e16, jnp.full((L,), jj, jnp.int32))
            dlj = dl16[jj]
            do = pl.multiple_of(dlj * L, L)
            denloc[pl.ds(do, L)] = denloc[pl.ds(do, L)] + ej
            nrow = numloc.at[dlj]
            vrow = vv.at[bs, j]
            for t in range(D // L):
              nrow[pl.ds(t * L, L)] = (nrow[pl.ds(t * L, L)]
                                       + vrow[pl.ds(t * L, L)] * ej)
          return 0

        lax.fori_loop(0, BV // L, group, 0)

      def chunk(i, _):
        @pl.when(i + 1 < nch)
        def _():
          fetch_chunk(i + 1, (i + 1) % 2)

        @pl.when(i % 2 == 0)
        def _():
          do_chunk(i, 0)

        @pl.when(i % 2 == 1)
        def _():
          do_chunk(i, 1)

        return 0

      lax.fori_loop(0, nch, chunk, 0)
      return 0

    lax.fori_loop(0, NW, scanner, 0)
    pltpu.sync_copy(numloc, num_hbm.at[pl.ds(wid * NPT, NPT)])
    pltpu.sync_copy(denloc, den_hbm.at[pl.ds(wid * NPT * L, NPT * L)])

  return sc_accum


def _tc_qkvs(x, Wq, Wk, Wv, Ws, bq, bk, bv, bs):
  def body(xr, wqr, wkr, wvr, wsr, bqr, bkr, bvr, bsr, qo, ko, vo, so):
    xb = xr[...]
    qo[...] = jnp.dot(xb, wqr[...], preferred_element_type=jnp.float32) + bqr[...]
    ko[...] = jnp.dot(xb, wkr[...], preferred_element_type=jnp.float32) + bkr[...]
    vo[...] = jnp.dot(xb, wvr[...], preferred_element_type=jnp.float32) + bvr[...]
    so[...] = jnp.dot(xb, wsr[...], preferred_element_type=jnp.float32) + bsr[...]

  return pl.pallas_call(
      body,
      grid=(GRID,),
      in_specs=[pl.BlockSpec((RB, D), lambda i: (i, 0))]
      + [pl.BlockSpec((D, D), lambda i: (0, 0))] * 4
      + [pl.BlockSpec((1, D), lambda i: (0, 0))] * 4,
      out_specs=[pl.BlockSpec((RB, D), lambda i: (i, 0))] * 4,
      out_shape=[jax.ShapeDtypeStruct((NP, D), jnp.float32)] * 4,
  )(x, Wq, Wk, Wv, Ws, bq.reshape(1, D), bk.reshape(1, D),
    bv.reshape(1, D), bs.reshape(1, D))


def _tc_combine(num, den, skip, do_relu):
  def body(numr, denr, skipr, outr):
    d = denr[:, 0] + 1e-16
    o = numr[...] / d[:, None] + skipr[...]
    if do_relu:
      o = jnp.maximum(o, 0.0)
    outr[...] = o

  return pl.pallas_call(
      body,
      grid=(GRID,),
      in_specs=[pl.BlockSpec((RB, D), lambda i: (i, 0)),
                pl.BlockSpec((RB, L), lambda i: (i, 0)),
                pl.BlockSpec((RB, D), lambda i: (i, 0))],
      out_specs=pl.BlockSpec((RB, D), lambda i: (i, 0)),
      out_shape=jax.ShapeDtypeStruct((NP, D), jnp.float32),
  )(num, den, skip)


def kernel(x, edge_index, Wq0, Wk0, Wv0, Ws0, Wq1, Wk1, Wv1, Ws1,
           bq0, bk0, bv0, bs0, bq1, bk1, bv1, bs1):
  xp = jnp.pad(x, ((0, NP - N), (0, 0)))
  src = edge_index[0]
  dst = edge_index[1]
  sc_alpha = _build_sc_alpha()
  sc_partition = _build_sc_partition()
  sc_accum = _build_sc_accum()

  psrc, pdl, peid, cnts = sc_partition(src, dst)

  q0, k0, v0, s0 = _tc_qkvs(xp, Wq0, Wk0, Wv0, Ws0, bq0, bk0, bv0, bs0)
  alpha0, mx0 = sc_alpha(src, dst, q0, k0)
  num0, den0 = sc_accum(psrc, pdl, peid, cnts, mx0, alpha0, v0)
  h = _tc_combine(num0, den0.reshape(NP, L), s0, True)

  q1, k1, v1, s1 = _tc_qkvs(h, Wq1, Wk1, Wv1, Ws1, bq1, bk1, bv1, bs1)
  alpha1, mx1 = sc_alpha(src, dst, q1, k1)
  num1, den1 = sc_accum(psrc, pdl, peid, cnts, mx1, alpha1, v1)
  out = _tc_combine(num1, den1.reshape(NP, L), s1, False)
  return out[:N]


# speedup vs baseline: 1.6444x; 1.6444x over previous
import functools

import jax
import jax.numpy as jnp
from jax import lax
from jax.experimental import pallas as pl
from jax.experimental.pallas import tpu as pltpu
from jax.experimental.pallas import tpu_sc as plsc

N = 10000
E = 320000
D = 128
NP = 10240
NC = 2
NS = 16
L = 16
NW = NC * NS
EPW = E // NW
BE = 80
NCHUNK = EPW // BE
NPT = NP // NW
BKT = 512
BKE = BKT - 1
BEC = 2000
NCH_P = EPW // BEC
BV = 64
RB = 512
GRID = NP // RB
RSQRT_C = 1.0 / (128.0 ** 0.5)

_mesh = plsc.VectorSubcoreMesh(core_axis_name="c", subcore_axis_name="s")


def _permute(v, idx):
  return v.at[idx].get(mode="promise_in_bounds")


def _splat_sum(v, lane):
  for sh in (8, 4, 2, 1):
    v = v + _permute(v, lane ^ sh)
  return v


def _splat_max(v, lane):
  for sh in (8, 4, 2, 1):
    v = jnp.maximum(v, _permute(v, lane ^ sh))
  return v


def _build_sc_alpha():
  @functools.partial(
      pl.kernel,
      out_type=[jax.ShapeDtypeStruct((E,), jnp.float32),
                jax.ShapeDtypeStruct((NW, L), jnp.float32)],
      mesh=_mesh,
      scratch_types=[
          pltpu.VMEM((2, BE), jnp.int32),
          pltpu.VMEM((2, BE), jnp.int32),
          pltpu.VMEM((2, BE, D), jnp.float32),
          pltpu.VMEM((2, BE, D), jnp.float32),
          pltpu.VMEM((BE,), jnp.float32),
          pltpu.VMEM((L,), jnp.float32),
          pltpu.SemaphoreType.DMA((2,)),
          pltpu.SemaphoreType.DMA((2,)),
      ],
  )
  def sc_alpha(src_hbm, dst_hbm, q_hbm, k_hbm, alpha_hbm, mx_hbm,
               srcv, dstv, qv, kv, av, mxv, semq, semk):
    c = lax.axis_index("c")
    s = lax.axis_index("s")
    wid = s * NC + c
    base = wid * EPW

    def fetch(i, b):
      off = base + i * BE
      pltpu.sync_copy(src_hbm.at[pl.ds(off, BE)], srcv.at[b])
      pltpu.sync_copy(dst_hbm.at[pl.ds(off, BE)], dstv.at[b])
      pltpu.async_copy(q_hbm.at[dstv.at[b]], qv.at[b], semq.at[b])
      pltpu.async_copy(k_hbm.at[srcv.at[b]], kv.at[b], semk.at[b])

    fetch(0, 0)
    mxv[...] = jnp.full((L,), -jnp.inf, jnp.float32)
    lane = lax.iota(jnp.int32, L)

    def compute(bs):
      def group(g, _):
        alpha16 = jnp.zeros((L,), jnp.float32)
        for jj in range(L):
          j = g * L + jj
          qrow = qv.at[bs, j]
          krow = kv.at[bs, j]
          acc = qrow[pl.ds(0, L)] * krow[pl.ds(0, L)]
          for t in range(1, D // L):
            acc = acc + qrow[pl.ds(t * L, L)] * krow[pl.ds(t * L, L)]
          a = _splat_sum(acc, lane) * RSQRT_C
          alpha16 = jnp.where(lane == jj, a, alpha16)
        av[pl.ds(g * L, L)] = alpha16
        mxv[...] = jnp.maximum(mxv[...], alpha16)
        return 0

      lax.fori_loop(0, BE // L, group, 0)

    def chunk(i, _):
      b = i % 2

      @pl.when(i + 1 < NCHUNK)
      def _():
        fetch(i + 1, (i + 1) % 2)

      @pl.when(b == 0)
      def _():
        pltpu.make_async_copy(q_hbm.at[dstv.at[0]], qv.at[0],
                              semq.at[0]).wait()
        pltpu.make_async_copy(k_hbm.at[srcv.at[0]], kv.at[0],
                              semk.at[0]).wait()
        compute(0)

      @pl.when(b == 1)
      def _():
        pltpu.make_async_copy(q_hbm.at[dstv.at[1]], qv.at[1],
                              semq.at[1]).wait()
        pltpu.make_async_copy(k_hbm.at[srcv.at[1]], kv.at[1],
                              semk.at[1]).wait()
        compute(1)

      pltpu.sync_copy(av, alpha_hbm.at[pl.ds(base + i * BE, BE)])
      return 0

    lax.fori_loop(0, NCHUNK, chunk, 0)
    pltpu.sync_copy(mxv, mx_hbm.at[wid])

  return sc_alpha


def _build_sc_partition():
  @functools.partial(
      pl.kernel,
      out_type=[jax.ShapeDtypeStruct((NW, NW, BKT), jnp.int32),
                jax.ShapeDtypeStruct((NW, NW, BKT), jnp.int32),
                jax.ShapeDtypeStruct((NW, NW, BKT), jnp.int32),
                jax.ShapeDtypeStruct((NW * NW * L,), jnp.int32)],
      mesh=_mesh,
      scratch_types=[
          pltpu.VMEM((BEC,), jnp.int32),
          pltpu.VMEM((BEC,), jnp.int32),
          pltpu.VMEM((NW, BKT), jnp.int32),
          pltpu.VMEM((NW, BKT), jnp.int32),
          pltpu.VMEM((NW, BKT), jnp.int32),
          pltpu.VMEM((NW * L,), jnp.int32),
          pltpu.SMEM((NW,), jnp.int32),
      ],
  )
  def sc_partition(src_hbm, dst_hbm, psrc_hbm, pdl_hbm, peid_hbm, cnt_hbm,
                   srcv, dstv, bsrc, bdl, beid, cntv, cnts):
    c = lax.axis_index("c")
    s = lax.axis_index("s")
    wid = s * NC + c
    base = wid * EPW

    def zcnt(o, _):
      cnts[o] = 0
      return 0

    lax.fori_loop(0, NW, zcnt, 0)

    def zfill(i, _):
      r = i // (BKT // L)
      t = i % (BKT // L)
      z = jnp.zeros((L,), jnp.int32)
      bsrc.at[r][pl.ds(t * L, L)] = z
      bdl.at[r][pl.ds(t * L, L)] = z
      beid.at[r][pl.ds(t * L, L)] = z
      return 0

    lax.fori_loop(0, NW * (BKT // L), zfill, 0)

    def chunk(i, _):
      off = base + i * BEC
      pltpu.sync_copy(src_hbm.at[pl.ds(off, BEC)], srcv)
      pltpu.sync_copy(dst_hbm.at[pl.ds(off, BEC)], dstv)

      def group(g, _):
        d16 = dstv[pl.ds(g * L, L)]
        s16 = srcv[pl.ds(g * L, L)]
        lane = lax.iota(jnp.int32, L)
        for jj in range(L):
          dj = d16[jj]
          o = dj // NPT
          dl = dj - o * NPT
          cnt = jnp.minimum(cnts[o], BKE)
          eid = off + g * L + jj
          gb = pl.multiple_of((cnt // L) * L, L)
          sel = lane == (cnt - gb)
          srow = bsrc.at[o]
          srow[pl.ds(gb, L)] = jnp.where(
              sel, jnp.full((L,), s16[jj], jnp.int32), srow[pl.ds(gb, L)])
          drow = bdl.at[o]
          drow[pl.ds(gb, L)] = jnp.where(
              sel, jnp.full((L,), dl, jnp.int32), drow[pl.ds(gb, L)])
          erow = beid.at[o]
          erow[pl.ds(gb, L)] = jnp.where(
              sel, jnp.full((L,), eid, jnp.int32), erow[pl.ds(gb, L)])
          cnts[o] = cnt + 1
        return 0

      lax.fori_loop(0, BEC // L, group, 0)
      return 0

    lax.fori_loop(0, NCH_P, chunk, 0)

    def emit(o, _):
      cntv[pl.ds(o * L, L)] = jnp.full(
          (L,), jnp.minimum(cnts[o], BKE), jnp.int32)
      return 0

    lax.fori_loop(0, NW, emit, 0)
    pltpu.sync_copy(bsrc, psrc_hbm.at[wid])
    pltpu.sync_copy(bdl, pdl_hbm.at[wid])
    pltpu.sync_copy(beid, peid_hbm.at[wid])
    pltpu.sync_copy(cntv, cnt_hbm.at[pl.ds(wid * NW * L, NW * L)])

  return sc_partition


def _build_sc_accum():
  @functools.partial(
      pl.kernel,
      out_type=[jax.ShapeDtypeStruct((NP, D), jnp.float32),
                jax.ShapeDtypeStruct((NP * L,), jnp.float32)],
      mesh=_mesh,
      scratch_types=[
          pltpu.VMEM((2 * BKT,), jnp.int32),
          pltpu.VMEM((2 * BKT,), jnp.int32),
          pltpu.VMEM((2 * BKT,), jnp.int32),
          pltpu.VMEM((NW * NW * L,), jnp.int32),
          pltpu.VMEM((NW, L), jnp.float32),
          pltpu.VMEM((2, BV, D), jnp.float32),
          pltpu.VMEM((2 * BV,), jnp.float32),
          pltpu.VMEM((NPT, D), jnp.float32),
          pltpu.VMEM((NPT * L,), jnp.float32),
          pltpu.SemaphoreType.DMA((2,)),
          pltpu.SemaphoreType.DMA((2,)),
          pltpu.SemaphoreType.DMA((2,)),
      ],
  )
  def sc_accum(psrc_hbm, pdl_hbm, peid_hbm, cnt_hbm, mx_hbm, alpha_hbm, v_hbm,
               num_hbm, den_hbm,
               csrc, cdl, ceid, cntv, mxall, vv, avb, numloc, denloc,
               semr, sema, semv):
    c = lax.axis_index("c")
    s = lax.axis_index("s")
    wid = s * NC + c
    lane = lax.iota(jnp.int32, L)

    pltpu.sync_copy(cnt_hbm, cntv)
    pltpu.sync_copy(mx_hbm, mxall)

    def mred(i, mv):
      return jnp.maximum(mv, mxall[i])

    mv = lax.fori_loop(0, NW, mred, jnp.full((L,), -jnp.inf, jnp.float32))
    m = _splat_max(mv, lane)

    def znum(i, _):
      r = i // (D // L)
      t = i % (D // L)
      numloc.at[r][pl.ds(t * L, L)] = jnp.zeros((L,), jnp.float32)
      return 0

    lax.fori_loop(0, NPT * (D // L), znum, 0)

    def zden(i, _):
      denloc[pl.ds(i * L, L)] = jnp.zeros((L,), jnp.float32)
      return 0

    lax.fori_loop(0, NPT, zden, 0)

    def fetch_rows(sc, b):
      bo = pl.multiple_of(b * BKT, BKT)
      pltpu.async_copy(psrc_hbm.at[sc, wid], csrc.at[pl.ds(bo, BKT)],
                       semr.at[b])
      pltpu.async_copy(pdl_hbm.at[sc, wid], cdl.at[pl.ds(bo, BKT)],
                       semr.at[b])
      pltpu.async_copy(peid_hbm.at[sc, wid], ceid.at[pl.ds(bo, BKT)],
                       semr.at[b])

    def wait_rows(sc, b):
      bo = pl.multiple_of(b * BKT, BKT)
      pltpu.make_async_copy(psrc_hbm.at[sc, wid], csrc.at[pl.ds(bo, BKT)],
                            semr.at[b]).wait()
      pltpu.make_async_copy(pdl_hbm.at[sc, wid], cdl.at[pl.ds(bo, BKT)],
                            semr.at[b]).wait()
      pltpu.make_async_copy(peid_hbm.at[sc, wid], ceid.at[pl.ds(bo, BKT)],
                            semr.at[b]).wait()

    fetch_rows(0, 0)

    def scanner(sc, _):
      br = sc % 2

      @pl.when(sc + 1 < NW)
      def _():
        fetch_rows(sc + 1, (sc + 1) % 2)

      wait_rows(sc, br)
      co = pl.multiple_of((sc * NW + wid) * L, L)
      cnt16 = cntv[pl.ds(co, L)]
      cnt = cnt16[0]
      nch = (cnt + (BV - 1)) // BV

      def fetch_chunk(i, b):
        ro = pl.multiple_of(br * BKT + i * BV, BV)
        ao = pl.multiple_of(b * BV, BV)
        pltpu.async_copy(alpha_hbm.at[ceid.at[pl.ds(ro, BV)]],
                         avb.at[pl.ds(ao, BV)], sema.at[b])
        pltpu.async_copy(v_hbm.at[csrc.at[pl.ds(ro, BV)]],
                         vv.at[b], semv.at[b])

      @pl.when(nch > 0)
      def _():
        fetch_chunk(0, 0)

      def do_chunk(i, bs):
        off = i * BV
        ro = pl.multiple_of(br * BKT + off, BV)
        pltpu.make_async_copy(alpha_hbm.at[ceid.at[pl.ds(ro, BV)]],
                              avb.at[pl.ds(bs * BV, BV)],
                              sema.at[bs]).wait()
        pltpu.make_async_copy(v_hbm.at[csrc.at[pl.ds(ro, BV)]],
                              vv.at[bs], semv.at[bs]).wait()

        def group(g, _):
          a16 = avb[pl.ds(bs * BV + g * L, L)]
          idx16 = jnp.full((L,), off + g * L, jnp.int32) + lane
          live = idx16 < cnt16
          e16 = jnp.where(live, jnp.exp(a16 - m), 0.0)
          dl16 = cdl[pl.ds(br * BKT + off + g * L, L)]
          for jj in range(L):
            j = g * L + jj
            ej = _permute(e16, jnp.full((L,), jj, jnp.int32))
            dlj = dl16[jj]
            do = pl.multiple_of(dlj * L, L)
            denloc[pl.ds(do, L)] = denloc[pl.ds(do, L)] + ej
            nrow = numloc.at[dlj]
            vrow = vv.at[bs, j]
            for t in range(D // L):
              nrow[pl.ds(t * L, L)] = (nrow[pl.ds(t * L, L)]
                                       + vrow[pl.ds(t * L, L)] * ej)
          return 0

        lax.fori_loop(0, BV // L, group, 0)

      def chunk(i, _):
        @pl.when(i + 1 < nch)
        def _():
          fetch_chunk(i + 1, (i + 1) % 2)

        @pl.when(i % 2 == 0)
        def _():
          do_chunk(i, 0)

        @pl.when(i % 2 == 1)
        def _():
          do_chunk(i, 1)

        return 0

      lax.fori_loop(0, nch, chunk, 0)
      return 0

    lax.fori_loop(0, NW, scanner, 0)
    pltpu.sync_copy(numloc, num_hbm.at[pl.ds(wid * NPT, NPT)])
    pltpu.sync_copy(denloc, den_hbm.at[pl.ds(wid * NPT * L, NPT * L)])

  return sc_accum


def _tc_qkvs(x, Wq, Wk, Wv, Ws, bq, bk, bv, bs):
  def body(xr, wqr, wkr, wvr, wsr, bqr, bkr, bvr, bsr, qo, ko, vo, so):
    xb = xr[...]
    qo[...] = jnp.dot(xb, wqr[...], preferred_element_type=jnp.float32) + bqr[...]
    ko[...] = jnp.dot(xb, wkr[...], preferred_element_type=jnp.float32) + bkr[...]
    vo[...] = jnp.dot(xb, wvr[...], preferred_element_type=jnp.float32) + bvr[...]
    so[...] = jnp.dot(xb, wsr[...], preferred_element_type=jnp.float32) + bsr[...]

  return pl.pallas_call(
      body,
      grid=(GRID,),
      in_specs=[pl.BlockSpec((RB, D), lambda i: (i, 0))]
      + [pl.BlockSpec((D, D), lambda i: (0, 0))] * 4
      + [pl.BlockSpec((1, D), lambda i: (0, 0))] * 4,
      out_specs=[pl.BlockSpec((RB, D), lambda i: (i, 0))] * 4,
      out_shape=[jax.ShapeDtypeStruct((NP, D), jnp.float32)] * 4,
  )(x, Wq, Wk, Wv, Ws, bq.reshape(1, D), bk.reshape(1, D),
    bv.reshape(1, D), bs.reshape(1, D))


def _tc_combine(num, den, skip, do_relu):
  def body(numr, denr, skipr, outr):
    d = denr[:, 0] + 1e-16
    o = numr[...] / d[:, None] + skipr[...]
    if do_relu:
      o = jnp.maximum(o, 0.0)
    outr[...] = o

  return pl.pallas_call(
      body,
      grid=(GRID,),
      in_specs=[pl.BlockSpec((RB, D), lambda i: (i, 0)),
                pl.BlockSpec((RB, L), lambda i: (i, 0)),
                pl.BlockSpec((RB, D), lambda i: (i, 0))],
      out_specs=pl.BlockSpec((RB, D), lambda i: (i, 0)),
      out_shape=jax.ShapeDtypeStruct((NP, D), jnp.float32),
  )(num, den, skip)


def kernel(x, edge_index, Wq0, Wk0, Wv0, Ws0, Wq1, Wk1, Wv1, Ws1,
           bq0, bk0, bv0, bs0, bq1, bk1, bv1, bs1):
  xp = jnp.pad(x, ((0, NP - N), (0, 0)))
  src = edge_index[0]
  dst = edge_index[1]
  sc_alpha = _build_sc_alpha()
  sc_partition = _build_sc_partition()
  sc_accum = _build_sc_accum()

  psrc, pdl, peid, cnts = sc_partition(src, dst)

  q0, k0, v0, s0 = _tc_qkvs(xp, Wq0, Wk0, Wv0, Ws0, bq0, bk0, bv0, bs0)
  alpha0, mx0 = sc_alpha(src, dst, q0, k0)
  num0, den0 = sc_accum(psrc, pdl, peid, cnts, mx0, alpha0, v0)
  h = _tc_combine(num0, den0.reshape(NP, L), s0, True)

  q1, k1, v1, s1 = _tc_qkvs(h, Wq1, Wk1, Wv1, Ws1, bq1, bk1, bv1, bs1)
  alpha1, mx1 = sc_alpha(src, dst, q1, k1)
  num1, den1 = sc_accum(psrc, pdl, peid, cnts, mx1, alpha1, v1)
  out = _tc_combine(num1, den1.reshape(NP, L), s1, False)
  return out[:N]
